# baseline (speedup 1.0000x reference)
import jax
import jax.numpy as jnp
from jax import lax
from jax.experimental import pallas as pl
from jax.experimental.pallas import tpu as pltpu
from jax.experimental.pallas import tpu_sc as plsc

VOCAB = 1000000
D_EMB = 64
D_MODEL = 128
B, S = 1024, 200
N_TOK = B * S

_INFO = plsc.get_sparse_core_info()
NC, NS = _INFO.num_cores, _INFO.num_subcores
NW = NC * NS
ROWS_PER_W = N_TOK // NW
GRP = 16
N_GRPS = ROWS_PER_W // GRP


def _gather_body(idx_hbm, table_hbm, out_hbm, idx_v, sem):
    wid = lax.axis_index("s") * NC + lax.axis_index("c")
    base = wid * ROWS_PER_W
    pltpu.sync_copy(idx_hbm.at[pl.ds(base, ROWS_PER_W)], idx_v)

    def grp(g, _):
        vec = idx_v[pl.ds(g * GRP, GRP)]
        gbase = base + g * GRP
        for k in range(GRP):
            pltpu.async_copy(table_hbm.at[pl.ds(vec[k], 1)],
                             out_hbm.at[pl.ds(gbase + k, 1)], sem)
        return 0

    lax.fori_loop(0, N_GRPS, grp, 0)
    pltpu.make_async_copy(
        table_hbm.at[pl.ds(0, ROWS_PER_W)],
        out_hbm.at[pl.ds(base, ROWS_PER_W)], sem).wait()


def _sc_gather(idx_flat, token_table):
    mesh = plsc.VectorSubcoreMesh(core_axis_name="c", subcore_axis_name="s")
    k = pl.kernel(
        _gather_body,
        mesh=mesh,
        out_type=jax.ShapeDtypeStruct((N_TOK, D_EMB), jnp.float32),
        scratch_types=[
            pltpu.VMEM((ROWS_PER_W,), jnp.int32),
            pltpu.SemaphoreType.DMA,
        ],
    )
    return k(idx_flat, token_table)


BB = 32


def _tc_body(tok_ref, W_ref, pos_ref, pos_W_ref, seg_ref, seg_W_ref,
             cb_ref, gamma_ref, beta_ref, out_ref):
    c = (jnp.dot(pos_ref[:], pos_W_ref[:],
                 preferred_element_type=jnp.float32)
         + jnp.dot(seg_ref[:], seg_W_ref[:],
                   preferred_element_type=jnp.float32)
         + cb_ref[:][None, :])
    y = jnp.dot(tok_ref[:], W_ref[:],
                preferred_element_type=jnp.float32)
    y = y.reshape(BB, S, D_MODEL) + c[None, :, :]
    mu = jnp.mean(y, axis=-1, keepdims=True)
    d = y - mu
    var = jnp.mean(d * d, axis=-1, keepdims=True)
    out_ref[:] = d * lax.rsqrt(var + 1e-5) * gamma_ref[:] + beta_ref[:]


def _tc_compute(tok, W, pos_seq, pos_W, seg_row, seg_W, cb, gamma, beta):
    grid = (B // BB,)
    rep2 = lambda shape: pl.BlockSpec(shape, lambda i: (0, 0))
    rep1 = lambda shape: pl.BlockSpec(shape, lambda i: (0,))
    return pl.pallas_call(
        _tc_body,
        grid=grid,
        in_specs=[
            pl.BlockSpec((BB * S, D_EMB), lambda i: (i, 0)),
            rep2((D_EMB, D_MODEL)),
            rep2((S, D_EMB)),
            rep2((D_EMB, D_MODEL)),
            rep2((1, D_EMB)),
            rep2((D_EMB, D_MODEL)),
            rep1((D_MODEL,)),
            rep1((D_MODEL,)),
            rep1((D_MODEL,)),
        ],
        out_specs=pl.BlockSpec((BB, S, D_MODEL), lambda i: (i, 0, 0)),
        out_shape=jax.ShapeDtypeStruct((B, S, D_MODEL), jnp.float32),
    )(tok, W, pos_seq, pos_W, seg_row, seg_W, cb, gamma, beta)


def kernel(token_table, W, b, pos_table, pos_W, pos_b, seg_table, seg_W,
           seg_b, gamma, beta, sequence):
    idx_flat = sequence.astype(jnp.int32).reshape(N_TOK)
    tok = _sc_gather(idx_flat, token_table)
    cb = b + pos_b + seg_b
    return _tc_compute(tok, W, pos_table[:S], pos_W, seg_table[0:1],
                       seg_W, cb, gamma, beta)


# speedup vs baseline: 4.5458x; 4.5458x over previous
import jax
import jax.numpy as jnp
from jax import lax
from jax.experimental import pallas as pl
from jax.experimental.pallas import tpu as pltpu
from jax.experimental.pallas import tpu_sc as plsc

VOCAB = 1000000
D_EMB = 64
D_MODEL = 128
B, S = 1024, 200
N_TOK = B * S

_INFO = plsc.get_sparse_core_info()
NC, NS = _INFO.num_cores, _INFO.num_subcores
NW = NC * NS
ROWS_PER_W = N_TOK // NW
CHUNK = 128
N_CHUNKS = ROWS_PER_W // CHUNK
IDX_ROWS = 56
NBUF = 4


def _gather_body(idx_hbm, table2_hbm, out_hbm, idx_v, rows_v, gsem, wsem):
    wid = lax.axis_index("s") * NC + lax.axis_index("c")
    base = wid * ROWS_PER_W
    pltpu.sync_copy(idx_hbm.at[wid], idx_v)

    def start_gather(c, slot):
        pltpu.async_copy(table2_hbm.at[idx_v.at[c]], rows_v.at[slot], gsem)

    def wait_gather(slot):
        pltpu.make_async_copy(
            table2_hbm.at[pl.ds(0, CHUNK)], rows_v.at[slot], gsem).wait()

    for p in range(NBUF - 1):
        start_gather(p, p)

    def step(c, _):
        slot = lax.rem(c, NBUF)

        @pl.when(c + NBUF - 1 < N_CHUNKS)
        def _():
            start_gather(c + NBUF - 1, lax.rem(c + NBUF - 1, NBUF))

        wait_gather(slot)
        pltpu.async_copy(rows_v.at[slot],
                         out_hbm.at[pl.ds(base + c * CHUNK, CHUNK)], wsem)

        @pl.when(c >= NBUF - 1)
        def _():
            pltpu.make_async_copy(
                table2_hbm.at[pl.ds(0, CHUNK)],
                out_hbm.at[pl.ds(0, CHUNK)], wsem).wait()

        return 0

    lax.fori_loop(0, N_CHUNKS, step, 0)
    pltpu.make_async_copy(
        table2_hbm.at[pl.ds(0, (NBUF - 1) * CHUNK)],
        out_hbm.at[pl.ds(0, (NBUF - 1) * CHUNK)], wsem).wait()


def _sc_gather(idxp, table2):
    mesh = plsc.VectorSubcoreMesh(core_axis_name="c", subcore_axis_name="s")
    k = pl.kernel(
        _gather_body,
        mesh=mesh,
        out_type=jax.ShapeDtypeStruct((N_TOK, 2 * D_EMB), jnp.float32),
        scratch_types=[
            pltpu.VMEM((IDX_ROWS, CHUNK), jnp.int32),
            pltpu.VMEM((NBUF, CHUNK, 2 * D_EMB), jnp.float32),
            pltpu.SemaphoreType.DMA,
            pltpu.SemaphoreType.DMA,
        ],
    )
    return k(idxp, table2)


BB = 32


def _tc_body(tok2_ref, seq_ref, W2_ref, pos_ref, pos_W_ref, seg_ref,
             seg_W_ref, cb_ref, gamma_ref, beta_ref, out_ref):
    c = (jnp.dot(pos_ref[:], pos_W_ref[:],
                 preferred_element_type=jnp.float32)
         + jnp.dot(seg_ref[:], seg_W_ref[:],
                   preferred_element_type=jnp.float32)
         + cb_ref[:][None, :])
    y2 = jnp.dot(tok2_ref[:], W2_ref[:],
                 preferred_element_type=jnp.float32)
    y_lo = lax.slice(y2, (0, 0), (BB * S, D_MODEL)).reshape(BB, S, D_MODEL)
    y_hi = lax.slice(y2, (0, D_MODEL),
                     (BB * S, 2 * D_MODEL)).reshape(BB, S, D_MODEL)
    parity = (seq_ref[:] & 1)[:, :, None]
    y = jnp.where(parity == 1, y_hi, y_lo)
    y = y + c[None, :, :]
    mu = jnp.mean(y, axis=-1, keepdims=True)
    d = y - mu
    var = jnp.mean(d * d, axis=-1, keepdims=True)
    out_ref[:] = d * lax.rsqrt(var + 1e-5) * gamma_ref[:] + beta_ref[:]


def _tc_compute(tok2, seq, W2, pos_seq, pos_W, seg_row, seg_W, cb,
                gamma, beta):
    grid = (B // BB,)
    rep2 = lambda shape: pl.BlockSpec(shape, lambda i: (0, 0))
    rep1 = lambda shape: pl.BlockSpec(shape, lambda i: (0,))
    return pl.pallas_call(
        _tc_body,
        grid=grid,
        in_specs=[
            pl.BlockSpec((BB * S, 2 * D_EMB), lambda i: (i, 0)),
            pl.BlockSpec((BB, S), lambda i: (i, 0)),
            rep2((2 * D_EMB, 2 * D_MODEL)),
            rep2((S, D_EMB)),
            rep2((D_EMB, D_MODEL)),
            rep2((1, D_EMB)),
            rep2((D_EMB, D_MODEL)),
            rep1((D_MODEL,)),
            rep1((D_MODEL,)),
            rep1((D_MODEL,)),
        ],
        out_specs=pl.BlockSpec((BB, S, D_MODEL), lambda i: (i, 0, 0)),
        out_shape=jax.ShapeDtypeStruct((B, S, D_MODEL), jnp.float32),
    )(tok2, seq, W2, pos_seq, pos_W, seg_row, seg_W, cb, gamma, beta)


def kernel(token_table, W, b, pos_table, pos_W, pos_b, seg_table, seg_W,
           seg_b, gamma, beta, sequence):
    seq = sequence.astype(jnp.int32)
    table2 = token_table.reshape(VOCAB // 2, 2 * D_EMB)
    idx3 = (seq >> 1).reshape(NW, N_CHUNKS, CHUNK)
    idxp = jnp.pad(idx3, ((0, 0), (0, IDX_ROWS - N_CHUNKS), (0, 0)))
    tok2 = _sc_gather(idxp, table2)
    zero = jnp.zeros((D_EMB, D_MODEL), jnp.float32)
    W2 = jnp.block([[W, zero], [zero, W]])
    cb = b + pos_b + seg_b
    return _tc_compute(tok2, seq, W2, pos_table[:S], pos_W, seg_table[0:1],
                       seg_W, cb, gamma, beta)


# speedup vs baseline: 4.6216x; 1.0167x over previous
import jax
import jax.numpy as jnp
from jax import lax
from jax.experimental import pallas as pl
from jax.experimental.pallas import tpu as pltpu
from jax.experimental.pallas import tpu_sc as plsc

VOCAB = 1000000
D_EMB = 64
D_MODEL = 128
B, S = 1024, 200
N_TOK = B * S

_INFO = plsc.get_sparse_core_info()
NC, NS = _INFO.num_cores, _INFO.num_subcores
NW = NC * NS
ROWS_PER_W = N_TOK // NW
CHUNK = 128
N_CHUNKS = ROWS_PER_W // CHUNK
IDX_ROWS = 56
NBUF = 4


def _gather_body(idx_hbm, table2_hbm, out_hbm, idx_v, rows_v, gsem, wsem):
    wid = lax.axis_index("s") * NC + lax.axis_index("c")
    base = wid * ROWS_PER_W
    pltpu.sync_copy(idx_hbm.at[wid], idx_v)

    def start_gather(c, slot):
        pltpu.async_copy(table2_hbm.at[idx_v.at[c]], rows_v.at[slot], gsem)

    def wait_gather(slot):
        pltpu.make_async_copy(
            table2_hbm.at[pl.ds(0, CHUNK)], rows_v.at[slot], gsem).wait()

    for p in range(NBUF - 1):
        start_gather(p, p)

    def step(c, _):
        slot = lax.rem(c, NBUF)

        @pl.when(c + NBUF - 1 < N_CHUNKS)
        def _():
            start_gather(c + NBUF - 1, lax.rem(c + NBUF - 1, NBUF))

        wait_gather(slot)
        pltpu.async_copy(rows_v.at[slot],
                         out_hbm.at[pl.ds(base + c * CHUNK, CHUNK)], wsem)

        @pl.when(c >= NBUF - 1)
        def _():
            pltpu.make_async_copy(
                table2_hbm.at[pl.ds(0, CHUNK)],
                out_hbm.at[pl.ds(0, CHUNK)], wsem).wait()

        return 0

    lax.fori_loop(0, N_CHUNKS, step, 0)
    pltpu.make_async_copy(
        table2_hbm.at[pl.ds(0, (NBUF - 1) * CHUNK)],
        out_hbm.at[pl.ds(0, (NBUF - 1) * CHUNK)], wsem).wait()


def _sc_gather(idxp, table2):
    mesh = plsc.VectorSubcoreMesh(core_axis_name="c", subcore_axis_name="s")
    k = pl.kernel(
        _gather_body,
        mesh=mesh,
        out_type=jax.ShapeDtypeStruct((N_TOK, 2 * D_EMB), jnp.float32),
        scratch_types=[
            pltpu.VMEM((IDX_ROWS, CHUNK), jnp.int32),
            pltpu.VMEM((NBUF, CHUNK, 2 * D_EMB), jnp.float32),
            pltpu.SemaphoreType.DMA,
            pltpu.SemaphoreType.DMA,
        ],
    )
    return k(idxp, table2)


HV = VOCAB // 2
RB = 4000


def _pack_body(a_ref, b_ref, out_ref):
    out_ref[:, 0:D_EMB] = a_ref[:]
    out_ref[:, D_EMB:2 * D_EMB] = b_ref[:]


def _pack_table(token_table):
    grid = (HV // RB,)
    return pl.pallas_call(
        _pack_body,
        grid=grid,
        in_specs=[
            pl.BlockSpec((RB, D_EMB), lambda i: (i, 0)),
            pl.BlockSpec((RB, D_EMB), lambda i: (i + HV // RB, 0)),
        ],
        out_specs=pl.BlockSpec((RB, 2 * D_EMB), lambda i: (i, 0)),
        out_shape=jax.ShapeDtypeStruct((HV, 2 * D_EMB), jnp.float32),
    )(token_table, token_table)


BB = 32


def _tc_body(tok2_ref, seq_ref, W2_ref, pos_ref, pos_W_ref, seg_ref,
             seg_W_ref, cb_ref, gamma_ref, beta_ref, out_ref):
    c = (jnp.dot(pos_ref[:], pos_W_ref[:],
                 preferred_element_type=jnp.float32)
         + jnp.dot(seg_ref[:], seg_W_ref[:],
                   preferred_element_type=jnp.float32)
         + cb_ref[:][None, :])
    y2 = jnp.dot(tok2_ref[:], W2_ref[:],
                 preferred_element_type=jnp.float32)
    y_lo = lax.slice(y2, (0, 0), (BB * S, D_MODEL)).reshape(BB, S, D_MODEL)
    y_hi = lax.slice(y2, (0, D_MODEL),
                     (BB * S, 2 * D_MODEL)).reshape(BB, S, D_MODEL)
    seq3 = seq_ref[:][:, :, None]
    y = jnp.where(seq3 >= HV, y_hi, y_lo)
    y = y + c[None, :, :]
    mu = jnp.mean(y, axis=-1, keepdims=True)
    d = y - mu
    var = jnp.mean(d * d, axis=-1, keepdims=True)
    out_ref[:] = d * lax.rsqrt(var + 1e-5) * gamma_ref[:] + beta_ref[:]


def _tc_compute(tok2, seq, W2, pos_seq, pos_W, seg_row, seg_W, cb,
                gamma, beta):
    grid = (B // BB,)
    rep2 = lambda shape: pl.BlockSpec(shape, lambda i: (0, 0))
    rep1 = lambda shape: pl.BlockSpec(shape, lambda i: (0,))
    return pl.pallas_call(
        _tc_body,
        grid=grid,
        in_specs=[
            pl.BlockSpec((BB * S, 2 * D_EMB), lambda i: (i, 0)),
            pl.BlockSpec((BB, S), lambda i: (i, 0)),
            rep2((2 * D_EMB, 2 * D_MODEL)),
            rep2((S, D_EMB)),
            rep2((D_EMB, D_MODEL)),
            rep2((1, D_EMB)),
            rep2((D_EMB, D_MODEL)),
            rep1((D_MODEL,)),
            rep1((D_MODEL,)),
            rep1((D_MODEL,)),
        ],
        out_specs=pl.BlockSpec((BB, S, D_MODEL), lambda i: (i, 0, 0)),
        out_shape=jax.ShapeDtypeStruct((B, S, D_MODEL), jnp.float32),
    )(tok2, seq, W2, pos_seq, pos_W, seg_row, seg_W, cb, gamma, beta)


def kernel(token_table, W, b, pos_table, pos_W, pos_b, seg_table, seg_W,
           seg_b, gamma, beta, sequence):
    seq = sequence.astype(jnp.int32)
    table2 = _pack_table(token_table)
    idx3 = jnp.where(seq >= HV, seq - HV, seq).reshape(NW, N_CHUNKS, CHUNK)
    idxp = jnp.pad(idx3, ((0, 0), (0, IDX_ROWS - N_CHUNKS), (0, 0)))
    tok2 = _sc_gather(idxp, table2)
    zero = jnp.zeros((D_EMB, D_MODEL), jnp.float32)
    W2 = jnp.block([[W, zero], [zero, W]])
    cb = b + pos_b + seg_b
    return _tc_compute(tok2, seq, W2, pos_table[:S], pos_W, seg_table[0:1],
                       seg_W, cb, gamma, beta)


# speedup vs baseline: 5.6532x; 1.2232x over previous
import jax
import jax.numpy as jnp
from jax import lax
from jax.experimental import pallas as pl
from jax.experimental.pallas import tpu as pltpu
from jax.experimental.pallas import tpu_sc as plsc

VOCAB = 1000000
D_EMB = 64
D_MODEL = 128
B, S = 1024, 200
N_TOK = B * S

_INFO = plsc.get_sparse_core_info()
NC, NS = _INFO.num_cores, _INFO.num_subcores
NW = NC * NS
ROWS_PER_W = N_TOK // NW
CHUNK = 128
N_CHUNKS = ROWS_PER_W // CHUNK
IDX_ROWS = 56
NBUF = 4


def _gather_body(idx_hbm, table2_hbm, out_hbm, idx_v, rows_v, gsem, wsem):
    wid = lax.axis_index("s") * NC + lax.axis_index("c")
    base = wid * ROWS_PER_W
    pltpu.sync_copy(idx_hbm.at[wid], idx_v)

    def start_gather(c, slot):
        pltpu.async_copy(table2_hbm.at[idx_v.at[c]], rows_v.at[slot], gsem)

    def wait_gather(slot):
        pltpu.make_async_copy(
            table2_hbm.at[pl.ds(0, CHUNK)], rows_v.at[slot], gsem).wait()

    for p in range(NBUF - 1):
        start_gather(p, p)

    def step(c, _):
        slot = lax.rem(c, NBUF)

        @pl.when(c + NBUF - 1 < N_CHUNKS)
        def _():
            start_gather(c + NBUF - 1, lax.rem(c + NBUF - 1, NBUF))

        wait_gather(slot)
        pltpu.async_copy(rows_v.at[slot],
                         out_hbm.at[pl.ds(base + c * CHUNK, CHUNK)], wsem)

        @pl.when(c >= NBUF - 1)
        def _():
            pltpu.make_async_copy(
                table2_hbm.at[pl.ds(0, CHUNK)],
                out_hbm.at[pl.ds(0, CHUNK)], wsem).wait()

        return 0

    lax.fori_loop(0, N_CHUNKS, step, 0)
    pltpu.make_async_copy(
        table2_hbm.at[pl.ds(0, (NBUF - 1) * CHUNK)],
        out_hbm.at[pl.ds(0, (NBUF - 1) * CHUNK)], wsem).wait()


def _sc_gather(idxp, table2):
    mesh = plsc.VectorSubcoreMesh(core_axis_name="c", subcore_axis_name="s")
    k = pl.kernel(
        _gather_body,
        mesh=mesh,
        out_type=jax.ShapeDtypeStruct((N_TOK, 2 * D_EMB), jnp.float32),
        scratch_types=[
            pltpu.VMEM((IDX_ROWS, CHUNK), jnp.int32),
            pltpu.VMEM((NBUF, CHUNK, 2 * D_EMB), jnp.float32),
            pltpu.SemaphoreType.DMA,
            pltpu.SemaphoreType.DMA,
        ],
    )
    return k(idxp, table2)


CB = 2048
NGROUP = CB // 256
PACK_GRID = -(-VOCAB // CB)
T2_ROWS = PACK_GRID * (CB // 2)


def _pack_body(tt_ref, out_ref):
    for g in range(NGROUP):
        a = tt_ref[:, g * 256:g * 256 + 128].T
        bb_ = tt_ref[:, g * 256 + 128:(g + 1) * 256].T
        out_ref[g * 128:(g + 1) * 128, 0:D_EMB] = a
        out_ref[g * 128:(g + 1) * 128, D_EMB:2 * D_EMB] = bb_


def _pack_table(token_table_t):
    return pl.pallas_call(
        _pack_body,
        grid=(PACK_GRID,),
        in_specs=[pl.BlockSpec((D_EMB, CB), lambda i: (0, i))],
        out_specs=pl.BlockSpec((CB // 2, 2 * D_EMB), lambda i: (i, 0)),
        out_shape=jax.ShapeDtypeStruct((T2_ROWS, 2 * D_EMB), jnp.float32),
    )(token_table_t)


BB = 32


def _tc_body(tok2_ref, seq_ref, W2_ref, pos_ref, pos_W_ref, seg_ref,
             seg_W_ref, cb_ref, gamma_ref, beta_ref, out_ref):
    c = (jnp.dot(pos_ref[:], pos_W_ref[:],
                 preferred_element_type=jnp.float32)
         + jnp.dot(seg_ref[:], seg_W_ref[:],
                   preferred_element_type=jnp.float32)
         + cb_ref[:][None, :])
    y2 = jnp.dot(tok2_ref[:], W2_ref[:],
                 preferred_element_type=jnp.float32)
    y_lo = lax.slice(y2, (0, 0), (BB * S, D_MODEL)).reshape(BB, S, D_MODEL)
    y_hi = lax.slice(y2, (0, D_MODEL),
                     (BB * S, 2 * D_MODEL)).reshape(BB, S, D_MODEL)
    seq3 = seq_ref[:][:, :, None]
    y = jnp.where(((seq3 >> 7) & 1) == 1, y_hi, y_lo)
    y = y + c[None, :, :]
    mu = jnp.mean(y, axis=-1, keepdims=True)
    d = y - mu
    var = jnp.mean(d * d, axis=-1, keepdims=True)
    out_ref[:] = d * lax.rsqrt(var + 1e-5) * gamma_ref[:] + beta_ref[:]


def _tc_compute(tok2, seq, W2, pos_seq, pos_W, seg_row, seg_W, cb,
                gamma, beta):
    grid = (B // BB,)
    rep2 = lambda shape: pl.BlockSpec(shape, lambda i: (0, 0))
    rep1 = lambda shape: pl.BlockSpec(shape, lambda i: (0,))
    return pl.pallas_call(
        _tc_body,
        grid=grid,
        in_specs=[
            pl.BlockSpec((BB * S, 2 * D_EMB), lambda i: (i, 0)),
            pl.BlockSpec((BB, S), lambda i: (i, 0)),
            rep2((2 * D_EMB, 2 * D_MODEL)),
            rep2((S, D_EMB)),
            rep2((D_EMB, D_MODEL)),
            rep2((1, D_EMB)),
            rep2((D_EMB, D_MODEL)),
            rep1((D_MODEL,)),
            rep1((D_MODEL,)),
            rep1((D_MODEL,)),
        ],
        out_specs=pl.BlockSpec((BB, S, D_MODEL), lambda i: (i, 0, 0)),
        out_shape=jax.ShapeDtypeStruct((B, S, D_MODEL), jnp.float32),
    )(tok2, seq, W2, pos_seq, pos_W, seg_row, seg_W, cb, gamma, beta)


def kernel(token_table, W, b, pos_table, pos_W, pos_b, seg_table, seg_W,
           seg_b, gamma, beta, sequence):
    seq = sequence.astype(jnp.int32)
    table2 = _pack_table(token_table.T)
    idx3 = (((seq >> 8) << 7) | (seq & 127)).reshape(NW, N_CHUNKS, CHUNK)
    idxp = jnp.pad(idx3, ((0, 0), (0, IDX_ROWS - N_CHUNKS), (0, 0)))
    tok2 = _sc_gather(idxp, table2)
    zero = jnp.zeros((D_EMB, D_MODEL), jnp.float32)
    W2 = jnp.block([[W, zero], [zero, W]])
    cb = b + pos_b + seg_b
    return _tc_compute(tok2, seq, W2, pos_table[:S], pos_W, seg_table[0:1],
                       seg_W, cb, gamma, beta)


# speedup vs baseline: 8.0268x; 1.4199x over previous
import jax
import jax.numpy as jnp
from jax import lax
from jax.experimental import pallas as pl
from jax.experimental.pallas import tpu as pltpu
from jax.experimental.pallas import tpu_sc as plsc

VOCAB = 1000000
D_EMB = 64
D_MODEL = 128
B, S = 1024, 200
N_TOK = B * S

_INFO = plsc.get_sparse_core_info()
NC, NS = _INFO.num_cores, _INFO.num_subcores
NW = NC * NS
ROWS_PER_W = N_TOK // NW
CHUNK = 128
N_CHUNKS = ROWS_PER_W // CHUNK
IDX_ROWS = 56
NBUF = 4


def _gather_body(idx_hbm, table2_hbm, out_hbm, idx_v, rows_v, gsem, wsem):
    wid = lax.axis_index("s") * NC + lax.axis_index("c")
    base = wid * ROWS_PER_W
    pltpu.sync_copy(idx_hbm.at[wid], idx_v)

    def start_gather(c, slot):
        pltpu.async_copy(table2_hbm.at[idx_v.at[c]], rows_v.at[slot], gsem)

    def wait_gather(slot):
        pltpu.make_async_copy(
            table2_hbm.at[pl.ds(0, CHUNK)], rows_v.at[slot], gsem).wait()

    for p in range(NBUF - 1):
        start_gather(p, p)

    def step(c, _):
        slot = lax.rem(c, NBUF)

        @pl.when(c + NBUF - 1 < N_CHUNKS)
        def _():
            start_gather(c + NBUF - 1, lax.rem(c + NBUF - 1, NBUF))

        wait_gather(slot)
        pltpu.async_copy(rows_v.at[slot],
                         out_hbm.at[pl.ds(base + c * CHUNK, CHUNK)], wsem)

        @pl.when(c >= NBUF - 1)
        def _():
            pltpu.make_async_copy(
                table2_hbm.at[pl.ds(0, CHUNK)],
                out_hbm.at[pl.ds(0, CHUNK)], wsem).wait()

        return 0

    lax.fori_loop(0, N_CHUNKS, step, 0)
    pltpu.make_async_copy(
        table2_hbm.at[pl.ds(0, (NBUF - 1) * CHUNK)],
        out_hbm.at[pl.ds(0, (NBUF - 1) * CHUNK)], wsem).wait()


def _sc_gather(idxp, table2):
    mesh = plsc.VectorSubcoreMesh(core_axis_name="c", subcore_axis_name="s")
    k = pl.kernel(
        _gather_body,
        mesh=mesh,
        out_type=jax.ShapeDtypeStruct((N_TOK, 2 * D_EMB), jnp.float32),
        scratch_types=[
            pltpu.VMEM((IDX_ROWS, CHUNK), jnp.int32),
            pltpu.VMEM((NBUF, CHUNK, 2 * D_EMB), jnp.float32),
            pltpu.SemaphoreType.DMA,
            pltpu.SemaphoreType.DMA,
        ],
    )
    return k(idxp, table2)


CB = 8192
NGROUP = CB // 256
PACK_GRID = -(-VOCAB // CB)
T2_ROWS = PACK_GRID * (CB // 2)


def _pack_body(tt_ref, out_ref):
    for g in range(NGROUP):
        a = tt_ref[:, g * 256:g * 256 + 128].T
        bb_ = tt_ref[:, g * 256 + 128:(g + 1) * 256].T
        out_ref[g * 128:(g + 1) * 128, 0:D_EMB] = a
        out_ref[g * 128:(g + 1) * 128, D_EMB:2 * D_EMB] = bb_


def _pack_table(token_table_t):
    return pl.pallas_call(
        _pack_body,
        grid=(PACK_GRID,),
        in_specs=[pl.BlockSpec((D_EMB, CB), lambda i: (0, i))],
        out_specs=pl.BlockSpec((CB // 2, 2 * D_EMB), lambda i: (i, 0)),
        out_shape=jax.ShapeDtypeStruct((T2_ROWS, 2 * D_EMB), jnp.float32),
    )(token_table_t)


BB = 32


def _tc_body(tok2_ref, seq_ref, W2_ref, pos_ref, pos_W_ref, seg_ref,
             seg_W_ref, cb_ref, gamma_ref, beta_ref, out_ref):
    c = (jnp.dot(pos_ref[:], pos_W_ref[:],
                 preferred_element_type=jnp.float32)
         + jnp.dot(seg_ref[:], seg_W_ref[:],
                   preferred_element_type=jnp.float32)
         + cb_ref[:][None, :])
    y2 = jnp.dot(tok2_ref[:], W2_ref[:],
                 preferred_element_type=jnp.float32)
    y_lo = lax.slice(y2, (0, 0), (BB * S, D_MODEL)).reshape(BB, S, D_MODEL)
    y_hi = lax.slice(y2, (0, D_MODEL),
                     (BB * S, 2 * D_MODEL)).reshape(BB, S, D_MODEL)
    seq3 = seq_ref[:][:, :, None]
    y = jnp.where(((seq3 >> 7) & 1) == 1, y_hi, y_lo)
    y = y + c[None, :, :]
    mu = jnp.mean(y, axis=-1, keepdims=True)
    d = y - mu
    var = jnp.mean(d * d, axis=-1, keepdims=True)
    out_ref[:] = d * lax.rsqrt(var + 1e-5) * gamma_ref[:] + beta_ref[:]


def _tc_compute(tok2, seq, W2, pos_seq, pos_W, seg_row, seg_W, cb,
                gamma, beta):
    grid = (B // BB,)
    rep2 = lambda shape: pl.BlockSpec(shape, lambda i: (0, 0))
    rep1 = lambda shape: pl.BlockSpec(shape, lambda i: (0,))
    return pl.pallas_call(
        _tc_body,
        grid=grid,
        in_specs=[
            pl.BlockSpec((BB * S, 2 * D_EMB), lambda i: (i, 0)),
            pl.BlockSpec((BB, S), lambda i: (i, 0)),
            rep2((2 * D_EMB, 2 * D_MODEL)),
            rep2((S, D_EMB)),
            rep2((D_EMB, D_MODEL)),
            rep2((1, D_EMB)),
            rep2((D_EMB, D_MODEL)),
            rep1((D_MODEL,)),
            rep1((D_MODEL,)),
            rep1((D_MODEL,)),
        ],
        out_specs=pl.BlockSpec((BB, S, D_MODEL), lambda i: (i, 0, 0)),
        out_shape=jax.ShapeDtypeStruct((B, S, D_MODEL), jnp.float32),
    )(tok2, seq, W2, pos_seq, pos_W, seg_row, seg_W, cb, gamma, beta)


def kernel(token_table, W, b, pos_table, pos_W, pos_b, seg_table, seg_W,
           seg_b, gamma, beta, sequence):
    seq = sequence.astype(jnp.int32)
    table2 = _pack_table(token_table.T)
    idx3 = (((seq >> 8) << 7) | (seq & 127)).reshape(NW, N_CHUNKS, CHUNK)
    idxp = jnp.pad(idx3, ((0, 0), (0, IDX_ROWS - N_CHUNKS), (0, 0)))
    tok2 = _sc_gather(idxp, table2)
    zero = jnp.zeros((D_EMB, D_MODEL), jnp.float32)
    W2 = jnp.block([[W, zero], [zero, W]])
    cb = b + pos_b + seg_b
    return _tc_compute(tok2, seq, W2, pos_table[:S], pos_W, seg_table[0:1],
                       seg_W, cb, gamma, beta)


# speedup vs baseline: 8.6547x; 1.0782x over previous
import jax
import jax.numpy as jnp
from jax import lax
from jax.experimental import pallas as pl
from jax.experimental.pallas import tpu as pltpu
from jax.experimental.pallas import tpu_sc as plsc

VOCAB = 1000000
D_EMB = 64
D_MODEL = 128
B, S = 1024, 200
N_TOK = B * S

_INFO = plsc.get_sparse_core_info()
NC, NS = _INFO.num_cores, _INFO.num_subcores
NW = NC * NS
ROWS_PER_W = N_TOK // NW
CHUNK = 128
N_CHUNKS = ROWS_PER_W // CHUNK
IDX_ROWS = 56
NBUF = 4


def _gather_body(idx_hbm, table2_hbm, out_hbm, idx_v, rows_v, gsem, wsem):
    wid = lax.axis_index("s") * NC + lax.axis_index("c")
    base = wid * ROWS_PER_W
    pltpu.sync_copy(idx_hbm.at[wid], idx_v)

    def start_gather(c, slot):
        pltpu.async_copy(table2_hbm.at[idx_v.at[c]], rows_v.at[slot], gsem)

    def wait_gather(slot):
        pltpu.make_async_copy(
            table2_hbm.at[pl.ds(0, CHUNK)], rows_v.at[slot], gsem).wait()

    for p in range(NBUF - 1):
        start_gather(p, p)

    def step(c, _):
        slot = lax.rem(c, NBUF)

        @pl.when(c + NBUF - 1 < N_CHUNKS)
        def _():
            start_gather(c + NBUF - 1, lax.rem(c + NBUF - 1, NBUF))

        wait_gather(slot)
        pltpu.async_copy(rows_v.at[slot],
                         out_hbm.at[pl.ds(base + c * CHUNK, CHUNK)], wsem)

        @pl.when(c >= NBUF - 1)
        def _():
            pltpu.make_async_copy(
                table2_hbm.at[pl.ds(0, CHUNK)],
                out_hbm.at[pl.ds(0, CHUNK)], wsem).wait()

        return 0

    lax.fori_loop(0, N_CHUNKS, step, 0)
    pltpu.make_async_copy(
        table2_hbm.at[pl.ds(0, (NBUF - 1) * CHUNK)],
        out_hbm.at[pl.ds(0, (NBUF - 1) * CHUNK)], wsem).wait()


def _sc_gather(idxp, table2):
    mesh = plsc.VectorSubcoreMesh(core_axis_name="c", subcore_axis_name="s")
    k = pl.kernel(
        _gather_body,
        mesh=mesh,
        out_type=jax.ShapeDtypeStruct((N_TOK, 2 * D_EMB), jnp.float32),
        scratch_types=[
            pltpu.VMEM((IDX_ROWS, CHUNK), jnp.int32),
            pltpu.VMEM((NBUF, CHUNK, 2 * D_EMB), jnp.float32),
            pltpu.SemaphoreType.DMA,
            pltpu.SemaphoreType.DMA,
        ],
    )
    return k(idxp, table2)


CB = 16384
NGROUP = CB // 256
PACK_GRID = -(-VOCAB // CB)
T2_ROWS = PACK_GRID * (CB // 2)


def _pack_body(tt_ref, out_ref):
    for g in range(NGROUP):
        a = tt_ref[:, g * 256:g * 256 + 128].T
        bb_ = tt_ref[:, g * 256 + 128:(g + 1) * 256].T
        out_ref[g * 128:(g + 1) * 128, 0:D_EMB] = a
        out_ref[g * 128:(g + 1) * 128, D_EMB:2 * D_EMB] = bb_


def _pack_table(token_table_t):
    return pl.pallas_call(
        _pack_body,
        grid=(PACK_GRID,),
        in_specs=[pl.BlockSpec((D_EMB, CB), lambda i: (0, i))],
        out_specs=pl.BlockSpec((CB // 2, 2 * D_EMB), lambda i: (i, 0)),
        out_shape=jax.ShapeDtypeStruct((T2_ROWS, 2 * D_EMB), jnp.float32),
    )(token_table_t)


BB = 32


def _tc_body(tok2_ref, seq_ref, W2_ref, pos_ref, pos_W_ref, seg_ref,
             seg_W_ref, cb_ref, gamma_ref, beta_ref, out_ref):
    c = (jnp.dot(pos_ref[:], pos_W_ref[:],
                 preferred_element_type=jnp.float32)
         + jnp.dot(seg_ref[:], seg_W_ref[:],
                   preferred_element_type=jnp.float32)
         + cb_ref[:][None, :])
    y2 = jnp.dot(tok2_ref[:], W2_ref[:],
                 preferred_element_type=jnp.float32)
    y_lo = lax.slice(y2, (0, 0), (BB * S, D_MODEL)).reshape(BB, S, D_MODEL)
    y_hi = lax.slice(y2, (0, D_MODEL),
                     (BB * S, 2 * D_MODEL)).reshape(BB, S, D_MODEL)
    seq3 = seq_ref[:][:, :, None]
    y = jnp.where(((seq3 >> 7) & 1) == 1, y_hi, y_lo)
    y = y + c[None, :, :]
    mu = jnp.mean(y, axis=-1, keepdims=True)
    d = y - mu
    var = jnp.mean(d * d, axis=-1, keepdims=True)
    out_ref[:] = d * lax.rsqrt(var + 1e-5) * gamma_ref[:] + beta_ref[:]


def _tc_compute(tok2, seq, W2, pos_seq, pos_W, seg_row, seg_W, cb,
                gamma, beta):
    grid = (B // BB,)
    rep2 = lambda shape: pl.BlockSpec(shape, lambda i: (0, 0))
    rep1 = lambda shape: pl.BlockSpec(shape, lambda i: (0,))
    return pl.pallas_call(
        _tc_body,
        grid=grid,
        in_specs=[
            pl.BlockSpec((BB * S, 2 * D_EMB), lambda i: (i, 0)),
            pl.BlockSpec((BB, S), lambda i: (i, 0)),
            rep2((2 * D_EMB, 2 * D_MODEL)),
            rep2((S, D_EMB)),
            rep2((D_EMB, D_MODEL)),
            rep2((1, D_EMB)),
            rep2((D_EMB, D_MODEL)),
            rep1((D_MODEL,)),
            rep1((D_MODEL,)),
            rep1((D_MODEL,)),
        ],
        out_specs=pl.BlockSpec((BB, S, D_MODEL), lambda i: (i, 0, 0)),
        out_shape=jax.ShapeDtypeStruct((B, S, D_MODEL), jnp.float32),
    )(tok2, seq, W2, pos_seq, pos_W, seg_row, seg_W, cb, gamma, beta)


def kernel(token_table, W, b, pos_table, pos_W, pos_b, seg_table, seg_W,
           seg_b, gamma, beta, sequence):
    seq = sequence.astype(jnp.int32)
    table2 = _pack_table(token_table.T)
    idx3 = (((seq >> 8) << 7) | (seq & 127)).reshape(NW, N_CHUNKS, CHUNK)
    idxp = jnp.pad(idx3, ((0, 0), (0, IDX_ROWS - N_CHUNKS), (0, 0)))
    tok2 = _sc_gather(idxp, table2)
    zero = jnp.zeros((D_EMB, D_MODEL), jnp.float32)
    W2 = jnp.block([[W, zero], [zero, W]])
    cb = b + pos_b + seg_b
    return _tc_compute(tok2, seq, W2, pos_table[:S], pos_W, seg_table[0:1],
                       seg_W, cb, gamma, beta)


# speedup vs baseline: 9.1049x; 1.0520x over previous
import jax
import jax.numpy as jnp
from jax import lax
from jax.experimental import pallas as pl
from jax.experimental.pallas import tpu as pltpu
from jax.experimental.pallas import tpu_sc as plsc

VOCAB = 1000000
D_EMB = 64
D_MODEL = 128
B, S = 1024, 200
N_TOK = B * S

_INFO = plsc.get_sparse_core_info()
NC, NS = _INFO.num_cores, _INFO.num_subcores
NW = NC * NS
ROWS_PER_W = N_TOK // NW
CHUNK = 128
N_CHUNKS = ROWS_PER_W // CHUNK
IDX_ROWS = 56
NBUF = 4


def _gather_body(idx_hbm, table2_hbm, out_hbm, idx_v, rows_v, gsem, wsem):
    wid = lax.axis_index("s") * NC + lax.axis_index("c")
    base = wid * ROWS_PER_W
    pltpu.sync_copy(idx_hbm.at[wid], idx_v)

    def start_gather(c, slot):
        pltpu.async_copy(table2_hbm.at[idx_v.at[c]], rows_v.at[slot], gsem)

    def wait_gather(slot):
        pltpu.make_async_copy(
            table2_hbm.at[pl.ds(0, CHUNK)], rows_v.at[slot], gsem).wait()

    for p in range(NBUF - 1):
        start_gather(p, p)

    def step(c, _):
        slot = lax.rem(c, NBUF)

        @pl.when(c + NBUF - 1 < N_CHUNKS)
        def _():
            start_gather(c + NBUF - 1, lax.rem(c + NBUF - 1, NBUF))

        wait_gather(slot)
        pltpu.async_copy(rows_v.at[slot],
                         out_hbm.at[pl.ds(base + c * CHUNK, CHUNK)], wsem)

        @pl.when(c >= NBUF - 1)
        def _():
            pltpu.make_async_copy(
                table2_hbm.at[pl.ds(0, CHUNK)],
                out_hbm.at[pl.ds(0, CHUNK)], wsem).wait()

        return 0

    lax.fori_loop(0, N_CHUNKS, step, 0)
    pltpu.make_async_copy(
        table2_hbm.at[pl.ds(0, (NBUF - 1) * CHUNK)],
        out_hbm.at[pl.ds(0, (NBUF - 1) * CHUNK)], wsem).wait()


def _sc_gather(idxp, table2):
    mesh = plsc.VectorSubcoreMesh(core_axis_name="c", subcore_axis_name="s")
    k = pl.kernel(
        _gather_body,
        mesh=mesh,
        out_type=jax.ShapeDtypeStruct((N_TOK, 2 * D_EMB), jnp.float32),
        scratch_types=[
            pltpu.VMEM((IDX_ROWS, CHUNK), jnp.int32),
            pltpu.VMEM((NBUF, CHUNK, 2 * D_EMB), jnp.float32),
            pltpu.SemaphoreType.DMA,
            pltpu.SemaphoreType.DMA,
        ],
    )
    return k(idxp, table2)


CB = 32768
NGROUP = CB // 256
PACK_GRID = -(-VOCAB // CB)
T2_ROWS = PACK_GRID * (CB // 2)


def _pack_body(tt_ref, out_ref):
    for g in range(NGROUP):
        a = tt_ref[:, g * 256:g * 256 + 128].T
        bb_ = tt_ref[:, g * 256 + 128:(g + 1) * 256].T
        out_ref[g * 128:(g + 1) * 128, 0:D_EMB] = a
        out_ref[g * 128:(g + 1) * 128, D_EMB:2 * D_EMB] = bb_


def _pack_table(token_table_t):
    return pl.pallas_call(
        _pack_body,
        grid=(PACK_GRID,),
        in_specs=[pl.BlockSpec((D_EMB, CB), lambda i: (0, i))],
        out_specs=pl.BlockSpec((CB // 2, 2 * D_EMB), lambda i: (i, 0)),
        out_shape=jax.ShapeDtypeStruct((T2_ROWS, 2 * D_EMB), jnp.float32),
    )(token_table_t)


BB = 64


def _tc_body(tok2_ref, seq_ref, W2_ref, pos_ref, pos_W_ref, seg_ref,
             seg_W_ref, cb_ref, gamma_ref, beta_ref, out_ref):
    c = (jnp.dot(pos_ref[:], pos_W_ref[:],
                 preferred_element_type=jnp.float32)
         + jnp.dot(seg_ref[:], seg_W_ref[:],
                   preferred_element_type=jnp.float32)
         + cb_ref[:][None, :])
    y2 = jnp.dot(tok2_ref[:], W2_ref[:],
                 preferred_element_type=jnp.float32)
    y_lo = lax.slice(y2, (0, 0), (BB * S, D_MODEL)).reshape(BB, S, D_MODEL)
    y_hi = lax.slice(y2, (0, D_MODEL),
                     (BB * S, 2 * D_MODEL)).reshape(BB, S, D_MODEL)
    seq3 = seq_ref[:][:, :, None]
    y = jnp.where(((seq3 >> 7) & 1) == 1, y_hi, y_lo)
    y = y + c[None, :, :]
    mu = jnp.mean(y, axis=-1, keepdims=True)
    d = y - mu
    var = jnp.mean(d * d, axis=-1, keepdims=True)
    out_ref[:] = d * lax.rsqrt(var + 1e-5) * gamma_ref[:] + beta_ref[:]


def _tc_compute(tok2, seq, W2, pos_seq, pos_W, seg_row, seg_W, cb,
                gamma, beta):
    grid = (B // BB,)
    rep2 = lambda shape: pl.BlockSpec(shape, lambda i: (0, 0))
    rep1 = lambda shape: pl.BlockSpec(shape, lambda i: (0,))
    return pl.pallas_call(
        _tc_body,
        grid=grid,
        in_specs=[
            pl.BlockSpec((BB * S, 2 * D_EMB), lambda i: (i, 0)),
            pl.BlockSpec((BB, S), lambda i: (i, 0)),
            rep2((2 * D_EMB, 2 * D_MODEL)),
            rep2((S, D_EMB)),
            rep2((D_EMB, D_MODEL)),
            rep2((1, D_EMB)),
            rep2((D_EMB, D_MODEL)),
            rep1((D_MODEL,)),
            rep1((D_MODEL,)),
            rep1((D_MODEL,)),
        ],
        out_specs=pl.BlockSpec((BB, S, D_MODEL), lambda i: (i, 0, 0)),
        out_shape=jax.ShapeDtypeStruct((B, S, D_MODEL), jnp.float32),
    )(tok2, seq, W2, pos_seq, pos_W, seg_row, seg_W, cb, gamma, beta)


def kernel(token_table, W, b, pos_table, pos_W, pos_b, seg_table, seg_W,
           seg_b, gamma, beta, sequence):
    seq = sequence.astype(jnp.int32)
    table2 = _pack_table(token_table.T)
    idx3 = (((seq >> 8) << 7) | (seq & 127)).reshape(NW, N_CHUNKS, CHUNK)
    idxp = jnp.pad(idx3, ((0, 0), (0, IDX_ROWS - N_CHUNKS), (0, 0)))
    tok2 = _sc_gather(idxp, table2)
    zero = jnp.zeros((D_EMB, D_MODEL), jnp.float32)
    W2 = jnp.block([[W, zero], [zero, W]])
    cb = b + pos_b + seg_b
    return _tc_compute(tok2, seq, W2, pos_table[:S], pos_W, seg_table[0:1],
                       seg_W, cb, gamma, beta)


# speedup vs baseline: 10.7275x; 1.1782x over previous
import jax
import jax.numpy as jnp
from jax import lax
from jax.experimental import pallas as pl
from jax.experimental.pallas import tpu as pltpu
from jax.experimental.pallas import tpu_sc as plsc

VOCAB = 1000000
D_EMB = 64
D_MODEL = 128
B, S = 1024, 200
N_TOK = B * S

_INFO = plsc.get_sparse_core_info()
NC, NS = _INFO.num_cores, _INFO.num_subcores
NW = NC * NS
ROWS_PER_W = N_TOK // NW
CHUNK = 128
N_CHUNKS = ROWS_PER_W // CHUNK
IDX_ROWS = 56
NBUF = 4

CB = 32768
NGROUP = CB // 512
PACK_GRID = -(-VOCAB // CB)
T4_SLOTS = PACK_GRID * (CB // 4)


def _gather_body(idx_hbm, table4_hbm, out_hbm, idx_v, rows_v, gsem, wsem):
    wid = lax.axis_index("s") * NC + lax.axis_index("c")
    base = wid * ROWS_PER_W
    pltpu.sync_copy(idx_hbm.at[wid], idx_v)

    def start_gather(c, slot):
        pltpu.async_copy(table4_hbm.at[idx_v.at[c]], rows_v.at[slot], gsem)

    def wait_gather(slot):
        pltpu.make_async_copy(
            table4_hbm.at[pl.ds(0, CHUNK)], rows_v.at[slot], gsem).wait()

    for p in range(NBUF - 1):
        start_gather(p, p)

    def step(c, _):
        slot = lax.rem(c, NBUF)

        @pl.when(c + NBUF - 1 < N_CHUNKS)
        def _():
            start_gather(c + NBUF - 1, lax.rem(c + NBUF - 1, NBUF))

        wait_gather(slot)
        pltpu.async_copy(rows_v.at[slot],
                         out_hbm.at[pl.ds(base + c * CHUNK, CHUNK)], wsem)

        @pl.when(c >= NBUF - 1)
        def _():
            pltpu.make_async_copy(
                table4_hbm.at[pl.ds(0, CHUNK)],
                out_hbm.at[pl.ds(0, CHUNK)], wsem).wait()

        return 0

    lax.fori_loop(0, N_CHUNKS, step, 0)
    pltpu.make_async_copy(
        table4_hbm.at[pl.ds(0, (NBUF - 1) * CHUNK)],
        out_hbm.at[pl.ds(0, (NBUF - 1) * CHUNK)], wsem).wait()


def _sc_gather(idxp, table4):
    mesh = plsc.VectorSubcoreMesh(core_axis_name="c", subcore_axis_name="s")
    k = pl.kernel(
        _gather_body,
        mesh=mesh,
        out_type=jax.ShapeDtypeStruct((N_TOK, D_MODEL), jnp.int32),
        scratch_types=[
            pltpu.VMEM((IDX_ROWS, CHUNK), jnp.int32),
            pltpu.VMEM((NBUF, CHUNK, D_MODEL), jnp.int32),
            pltpu.SemaphoreType.DMA,
            pltpu.SemaphoreType.DMA,
        ],
    )
    return k(idxp, table4)


def _bf16_bits(x):
    return lax.convert_element_type(
        lax.bitcast_convert_type(x.astype(jnp.bfloat16), jnp.uint16),
        jnp.uint32)


def _pack_body(tt_ref, out_ref):
    for g in range(NGROUP):
        base = g * 512
        a = tt_ref[:, base:base + 128].T
        bq = tt_ref[:, base + 128:base + 256].T
        cq = tt_ref[:, base + 256:base + 384].T
        dq = tt_ref[:, base + 384:base + 512].T
        sl0 = jnp.concatenate([a, bq], axis=1)
        sl1 = jnp.concatenate([cq, dq], axis=1)
        packed = _bf16_bits(sl0) | (_bf16_bits(sl1) << 16)
        out_ref[g * 128:(g + 1) * 128, :] = lax.bitcast_convert_type(
            packed, jnp.int32)


def _pack_table(token_table_t):
    return pl.pallas_call(
        _pack_body,
        grid=(PACK_GRID,),
        in_specs=[pl.BlockSpec((D_EMB, CB), lambda i: (0, i))],
        out_specs=pl.BlockSpec((CB // 4, D_MODEL), lambda i: (i, 0)),
        out_shape=jax.ShapeDtypeStruct((T4_SLOTS, D_MODEL), jnp.int32),
    )(token_table_t)


BB = 64


def _tc_body(tok4_ref, seq_ref, W2_ref, pos_ref, pos_W_ref, seg_ref,
             seg_W_ref, cb_ref, gamma_ref, beta_ref, out_ref):
    c = (jnp.dot(pos_ref[:], pos_W_ref[:],
                 preferred_element_type=jnp.float32)
         + jnp.dot(seg_ref[:], seg_W_ref[:],
                   preferred_element_type=jnp.float32)
         + cb_ref[:][None, :])
    ti = lax.bitcast_convert_type(tok4_ref[:], jnp.uint32)
    ta = lax.bitcast_convert_type(
        lax.convert_element_type(ti & 0xFFFF, jnp.uint16), jnp.bfloat16)
    tb = lax.bitcast_convert_type(
        lax.convert_element_type(ti >> 16, jnp.uint16), jnp.bfloat16)
    ya = jnp.dot(ta, W2_ref[:], preferred_element_type=jnp.float32)
    yb = jnp.dot(tb, W2_ref[:], preferred_element_type=jnp.float32)
    y0 = lax.slice(ya, (0, 0), (BB * S, D_MODEL)).reshape(BB, S, D_MODEL)
    y1 = lax.slice(ya, (0, D_MODEL),
                   (BB * S, 2 * D_MODEL)).reshape(BB, S, D_MODEL)
    y2 = lax.slice(yb, (0, 0), (BB * S, D_MODEL)).reshape(BB, S, D_MODEL)
    y3 = lax.slice(yb, (0, D_MODEL),
                   (BB * S, 2 * D_MODEL)).reshape(BB, S, D_MODEL)
    sel = (seq_ref[:][:, :, None] >> 7) & 3
    y = jnp.where(sel == 0, y0,
                  jnp.where(sel == 1, y1,
                            jnp.where(sel == 2, y2, y3)))
    y = y + c[None, :, :]
    mu = jnp.mean(y, axis=-1, keepdims=True)
    d = y - mu
    var = jnp.mean(d * d, axis=-1, keepdims=True)
    out_ref[:] = d * lax.rsqrt(var + 1e-5) * gamma_ref[:] + beta_ref[:]


def _tc_compute(tok4, seq, W2, pos_seq, pos_W, seg_row, seg_W, cb,
                gamma, beta):
    grid = (B // BB,)
    rep2 = lambda shape: pl.BlockSpec(shape, lambda i: (0, 0))
    rep1 = lambda shape: pl.BlockSpec(shape, lambda i: (0,))
    return pl.pallas_call(
        _tc_body,
        grid=grid,
        in_specs=[
            pl.BlockSpec((BB * S, D_MODEL), lambda i: (i, 0)),
            pl.BlockSpec((BB, S), lambda i: (i, 0)),
            rep2((2 * D_EMB, 2 * D_MODEL)),
            rep2((S, D_EMB)),
            rep2((D_EMB, D_MODEL)),
            rep2((1, D_EMB)),
            rep2((D_EMB, D_MODEL)),
            rep1((D_MODEL,)),
            rep1((D_MODEL,)),
            rep1((D_MODEL,)),
        ],
        out_specs=pl.BlockSpec((BB, S, D_MODEL), lambda i: (i, 0, 0)),
        out_shape=jax.ShapeDtypeStruct((B, S, D_MODEL), jnp.float32),
    )(tok4, seq, W2, pos_seq, pos_W, seg_row, seg_W, cb, gamma, beta)


def kernel(token_table, W, b, pos_table, pos_W, pos_b, seg_table, seg_W,
           seg_b, gamma, beta, sequence):
    seq = sequence.astype(jnp.int32)
    table4 = _pack_table(token_table.T)
    idx3 = (((seq >> 9) << 7) | (seq & 127)).reshape(NW, N_CHUNKS, CHUNK)
    idxp = jnp.pad(idx3, ((0, 0), (0, IDX_ROWS - N_CHUNKS), (0, 0)))
    tok4 = _sc_gather(idxp, table4)
    zero = jnp.zeros((D_EMB, D_MODEL), jnp.float32)
    W2 = jnp.block([[W, zero], [zero, W]])
    cb = b + pos_b + seg_b
    return _tc_compute(tok4, seq, W2, pos_table[:S], pos_W, seg_table[0:1],
                       seg_W, cb, gamma, beta)
